# baseline (speedup 1.0000x reference)
import jax
import jax.numpy as jnp
from jax import lax
from jax.experimental import pallas as pl
from jax.experimental.pallas import tpu as pltpu
from jax.experimental.pallas import tpu_sc as plsc

B = 16384
V1 = 100001
D = 16
F = 26
DNUM = 13
OUT_W = DNUM + F * D

NC = 2
NS = 16
NW = NC * NS
BPW = B // NW
CH = 128
NCH = BPW // CH
IDX_W = F * NCH * CH
TBL_STRIDE = D * V1


def _body(numericT, cats, tflat, out, idx_v, idxtmp, rowbufT, sem):
    wid = lax.axis_index("s") * NC + lax.axis_index("c")
    base = wid * BPW

    pltpu.sync_copy(cats.at[wid], idx_v)

    @pl.loop(0, NCH)
    def chunk(c):
        rowbase = base + c * CH
        coff = pl.multiple_of(c * CH, CH)
        pltpu.sync_copy(numericT.at[:, pl.ds(rowbase, CH)],
                        rowbufT.at[pl.ds(0, 16), :])

        @pl.loop(0, F)
        def per_table(t):
            ioff = t * NCH * CH + coff
            copies = []
            for d in range(D):
                dv = jnp.full((16,), d * V1, jnp.int32)

                @pl.loop(0, CH // 16)
                def addoff(i):
                    idxtmp[d, pl.ds(i * 16, 16)] = (
                        idx_v[pl.ds(ioff + i * 16, 16)] + dv)

                copies.append(pltpu.async_copy(
                    tflat.at[idxtmp.at[d]],
                    rowbufT.at[DNUM + t * D + d],
                    sem,
                ))
            for cp in copies:
                cp.wait()

        pltpu.sync_copy(rowbufT, out.at[:, pl.ds(rowbase, CH)])


@jax.jit
def _run(numericT, cats, tflat):
    kern = pl.kernel(
        _body,
        out_type=jax.ShapeDtypeStruct((OUT_W, B), jnp.float32),
        mesh=plsc.VectorSubcoreMesh(
            core_axis_name="c", subcore_axis_name="s",
            num_cores=NC, num_subcores=NS,
        ),
        scratch_types=[
            pltpu.VMEM((IDX_W,), jnp.int32),
            pltpu.VMEM((D, CH), jnp.int32),
            pltpu.VMEM((OUT_W, CH), jnp.float32),
            pltpu.SemaphoreType.DMA,
        ],
        compiler_params=pltpu.CompilerParams(use_tc_tiling_on_sc=False),
    )
    outT = kern(numericT, cats, tflat)
    return outT.T


def kernel(numeric, cat_0, cat_1, cat_2, cat_3, cat_4, cat_5, cat_6, cat_7, cat_8, cat_9, cat_10, cat_11, cat_12, cat_13, cat_14, cat_15, cat_16, cat_17, cat_18, cat_19, cat_20, cat_21, cat_22, cat_23, cat_24, cat_25, W_0, W_1, W_2, W_3, W_4, W_5, W_6, W_7, W_8, W_9, W_10, W_11, W_12, W_13, W_14, W_15, W_16, W_17, W_18, W_19, W_20, W_21, W_22, W_23, W_24, W_25):
    cats = (cat_0, cat_1, cat_2, cat_3, cat_4, cat_5, cat_6, cat_7, cat_8,
            cat_9, cat_10, cat_11, cat_12, cat_13, cat_14, cat_15, cat_16,
            cat_17, cat_18, cat_19, cat_20, cat_21, cat_22, cat_23, cat_24,
            cat_25)
    tables = (W_0, W_1, W_2, W_3, W_4, W_5, W_6, W_7, W_8, W_9, W_10, W_11,
              W_12, W_13, W_14, W_15, W_16, W_17, W_18, W_19, W_20, W_21,
              W_22, W_23, W_24, W_25)
    offs = (jnp.arange(F, dtype=jnp.int32) * TBL_STRIDE + 1)[:, None]
    cats_w = (
        (jnp.stack(cats, 0) + offs)
        .reshape(F, NW, NCH * CH)
        .transpose(1, 0, 2)
        .reshape(NW, IDX_W)
    )
    tflat = jnp.concatenate([W.T.reshape(-1) for W in tables])
    numericT = jnp.pad(numeric, ((0, 0), (0, 16 - DNUM))).T
    return _run(numericT, cats_w, tflat)


# speedup vs baseline: 3.1609x; 3.1609x over previous
import jax
import jax.numpy as jnp
from jax import lax
from jax.experimental import pallas as pl
from jax.experimental.pallas import tpu as pltpu
from jax.experimental.pallas import tpu_sc as plsc

B = 16384
V1 = 100001
D = 16
F = 26
DNUM = 13
OUT_W = DNUM + F * D

NC = 2
NS = 16
NW = NC * NS
BPW = B // NW
CH = 128
NCH = BPW // CH
IDX_W = F * NCH * CH


def _body(*refs):
    numericT = refs[0]
    cats = refs[1]
    tabs = refs[2:2 + F]
    out = refs[2 + F]
    idx_v, idxtmp, rowbufT, sem = refs[3 + F:]

    wid = lax.axis_index("s") * NC + lax.axis_index("c")
    base = wid * BPW

    pltpu.sync_copy(cats.at[wid], idx_v)

    @pl.loop(0, NCH)
    def chunk(c):
        rowbase = base + c * CH
        coff = pl.multiple_of(c * CH, CH)
        pltpu.sync_copy(numericT.at[:, pl.ds(rowbase, CH)],
                        rowbufT.at[pl.ds(0, 16), :])

        for t in range(F):
            ioff = t * NCH * CH + coff

            @pl.loop(0, D)
            def per_d(d):
                dv = jnp.full((16,), V1, jnp.int32) * d

                @pl.loop(0, CH // 16)
                def addoff(i):
                    idxtmp[d, pl.ds(i * 16, 16)] = (
                        idx_v[pl.ds(ioff + i * 16, 16)] + dv)

                pltpu.async_copy(
                    tabs[t].at[idxtmp.at[d]],
                    rowbufT.at[DNUM + t * D + d],
                    sem,
                )

            pltpu.make_async_copy(
                numericT.at[:, pl.ds(0, CH)],
                rowbufT.at[pl.ds(DNUM + t * D, 16), :],
                sem,
            ).wait()

        pltpu.sync_copy(rowbufT, out.at[:, pl.ds(rowbase, CH)])


@jax.jit
def _run(numericT, cats, tflats):
    kern = pl.kernel(
        _body,
        out_type=jax.ShapeDtypeStruct((OUT_W, B), jnp.float32),
        mesh=plsc.VectorSubcoreMesh(
            core_axis_name="c", subcore_axis_name="s",
            num_cores=NC, num_subcores=NS,
        ),
        scratch_types=[
            pltpu.VMEM((IDX_W,), jnp.int32),
            pltpu.VMEM((D, CH), jnp.int32),
            pltpu.VMEM((OUT_W, CH), jnp.float32),
            pltpu.SemaphoreType.DMA,
        ],
        compiler_params=pltpu.CompilerParams(use_tc_tiling_on_sc=False),
    )
    outT = kern(numericT, cats, *tflats)
    return outT.T


def kernel(numeric, cat_0, cat_1, cat_2, cat_3, cat_4, cat_5, cat_6, cat_7, cat_8, cat_9, cat_10, cat_11, cat_12, cat_13, cat_14, cat_15, cat_16, cat_17, cat_18, cat_19, cat_20, cat_21, cat_22, cat_23, cat_24, cat_25, W_0, W_1, W_2, W_3, W_4, W_5, W_6, W_7, W_8, W_9, W_10, W_11, W_12, W_13, W_14, W_15, W_16, W_17, W_18, W_19, W_20, W_21, W_22, W_23, W_24, W_25):
    cats = (cat_0, cat_1, cat_2, cat_3, cat_4, cat_5, cat_6, cat_7, cat_8,
            cat_9, cat_10, cat_11, cat_12, cat_13, cat_14, cat_15, cat_16,
            cat_17, cat_18, cat_19, cat_20, cat_21, cat_22, cat_23, cat_24,
            cat_25)
    tables = (W_0, W_1, W_2, W_3, W_4, W_5, W_6, W_7, W_8, W_9, W_10, W_11,
              W_12, W_13, W_14, W_15, W_16, W_17, W_18, W_19, W_20, W_21,
              W_22, W_23, W_24, W_25)
    cats_w = (
        (jnp.stack(cats, 0) + 1)
        .reshape(F, NW, NCH * CH)
        .transpose(1, 0, 2)
        .reshape(NW, IDX_W)
    )
    tflats = [W.T.reshape(-1) for W in tables]
    numericT = jnp.pad(numeric, ((0, 0), (0, 16 - DNUM))).T
    return _run(numericT, cats_w, tflats)
